# baseline (speedup 1.0000x reference)
import functools

import jax
import jax.numpy as jnp
from jax import lax
from jax.experimental import pallas as pl
from jax.experimental.pallas import tpu as pltpu
from jax.experimental.pallas import tpu_sc as plsc

_NC = 2
_NS = 16
_NW = _NC * _NS

_ROWS = 4096
_NB = 8
_F = 128
_CH = 16
_PER_W = _ROWS // _NW
_NCHUNK = _PER_W // _CH


_SP = 3


def _sc_body(x1_hbm, x2_hbm, out_hbm, a_v, b_v, sa, sb, lsem, wsem, slsem,
             swsem):
    cid = lax.axis_index("c")
    sid = lax.axis_index("s")
    wid = sid * _NC + cid
    base = wid * _PER_W

    def start_loads(c):
        s = c % 2
        bk = base + c * _CH
        return [
            pltpu.async_copy(x1_hbm.at[pl.ds(bk, _CH)], a_v.at[s], lsem),
            pltpu.async_copy(x2_hbm.at[pl.ds(bk, _CH)], b_v.at[s], lsem),
            pltpu.async_copy(x1_hbm.at[pl.ds(bk, _CH)], sa.at[sid, s], slsem),
            pltpu.async_copy(x2_hbm.at[pl.ds(bk, _CH)], sb.at[sid, s], slsem),
        ]

    loads = {0: start_loads(0)}
    writes = {}
    for c in range(_NCHUNK):
        s = c % 2
        if c >= 1:
            for d in writes[c - 1]:
                d.wait()
        if c + 1 < _NCHUNK:
            loads[c + 1] = start_loads(c + 1)
        for d in loads[c]:
            d.wait()
        bk = base + c * _CH
        ws = []
        for i in range(_NB):
            if i < _SP:
                ws.append(pltpu.async_copy(
                    sa.at[sid, s],
                    out_hbm.at[pl.ds(bk, _CH), i, :, pl.ds(0, _F)], swsem))
            else:
                ws.append(pltpu.async_copy(
                    a_v.at[s],
                    out_hbm.at[pl.ds(bk, _CH), i, :, pl.ds(0, _F)], wsem))
        for j in range(_NB):
            if j < _SP:
                ws.append(pltpu.async_copy(
                    sb.at[sid, s],
                    out_hbm.at[pl.ds(bk, _CH), :, j, pl.ds(_F, _F)], swsem))
            else:
                ws.append(pltpu.async_copy(
                    b_v.at[s],
                    out_hbm.at[pl.ds(bk, _CH), :, j, pl.ds(_F, _F)], wsem))
        writes[c] = ws
    for d in writes[_NCHUNK - 1]:
        d.wait()


_sc_assemble = functools.partial(
    pl.kernel,
    out_type=jax.ShapeDtypeStruct((_ROWS, _NB, _NB, 2 * _F), jnp.float32),
    mesh=plsc.VectorSubcoreMesh(core_axis_name="c", subcore_axis_name="s"),
    scratch_types=[
        pltpu.VMEM((2, _CH, _NB, _F), jnp.float32),
        pltpu.VMEM((2, _CH, _NB, _F), jnp.float32),
        pltpu.VMEM_SHARED((_NS, 2, _CH, _NB, _F), jnp.float32),
        pltpu.VMEM_SHARED((_NS, 2, _CH, _NB, _F), jnp.float32),
        pltpu.SemaphoreType.DMA,
        pltpu.SemaphoreType.DMA,
        pltpu.SemaphoreType.DMA,
        pltpu.SemaphoreType.DMA,
    ],
)(_sc_body)


def kernel(x1, x2):
    nbatch, nk, nb, f = x1.shape
    rows = nbatch * nk
    out = _sc_assemble(x1.reshape(rows, nb, f), x2.reshape(rows, nb, f))
    return out.reshape(nbatch, nk, nb, nb, 2 * f)


# speedup vs baseline: 1.0678x; 1.0678x over previous
import functools

import jax
import jax.numpy as jnp
from jax import lax
from jax.experimental import pallas as pl
from jax.experimental.pallas import tpu as pltpu
from jax.experimental.pallas import tpu_sc as plsc

_NC = 2
_NS = 16
_NW = _NC * _NS

_ROWS = 4096
_NB = 8
_F = 128
_CH = 8
_SLOTS = 3
_PER_W = _ROWS // _NW
_NCHUNK = _PER_W // _CH


def _sc_body(x1_hbm, x2_hbm, out_hbm, a_v, b_v, lsem, wsem):
    wid = lax.axis_index("s") * _NC + lax.axis_index("c")
    base = wid * _PER_W

    def start_loads(c):
        s = c % _SLOTS
        bk = base + c * _CH
        return [
            pltpu.async_copy(x1_hbm.at[pl.ds(bk, _CH)], a_v.at[s], lsem),
            pltpu.async_copy(x2_hbm.at[pl.ds(bk, _CH)], b_v.at[s], lsem),
        ]

    loads = {c: start_loads(c) for c in range(_SLOTS - 1)}
    writes = {}
    for c in range(_NCHUNK):
        s = c % _SLOTS
        if c >= _SLOTS - 1:
            for d in writes[c - _SLOTS + 1]:
                d.wait()
        if c + _SLOTS - 1 < _NCHUNK:
            loads[c + _SLOTS - 1] = start_loads(c + _SLOTS - 1)
        for d in loads[c]:
            d.wait()
        bk = base + c * _CH
        ws = []
        for i in range(_NB):
            ws.append(pltpu.async_copy(
                a_v.at[s], out_hbm.at[pl.ds(bk, _CH), i, :, pl.ds(0, _F)],
                wsem))
            ws.append(pltpu.async_copy(
                b_v.at[s], out_hbm.at[pl.ds(bk, _CH), :, i, pl.ds(_F, _F)],
                wsem))
        writes[c] = ws
    for c in range(max(0, _NCHUNK - _SLOTS + 1), _NCHUNK):
        for d in writes[c]:
            d.wait()


_sc_assemble = functools.partial(
    pl.kernel,
    out_type=jax.ShapeDtypeStruct((_ROWS, _NB, _NB, 2 * _F), jnp.float32),
    mesh=plsc.VectorSubcoreMesh(core_axis_name="c", subcore_axis_name="s"),
    scratch_types=[
        pltpu.VMEM((_SLOTS, _CH, _NB, _F), jnp.float32),
        pltpu.VMEM((_SLOTS, _CH, _NB, _F), jnp.float32),
        pltpu.SemaphoreType.DMA,
        pltpu.SemaphoreType.DMA,
    ],
)(_sc_body)


def kernel(x1, x2):
    nbatch, nk, nb, f = x1.shape
    rows = nbatch * nk
    out = _sc_assemble(x1.reshape(rows, nb, f), x2.reshape(rows, nb, f))
    return out.reshape(nbatch, nk, nb, nb, 2 * f)
